# baseline (speedup 1.0000x reference)
import functools

import jax
import jax.numpy as jnp
from jax import lax
from jax.experimental import pallas as pl
from jax.experimental.pallas import tpu as pltpu
from jax.experimental.pallas import tpu_sc as plsc

N_FEATURES = 128
WINDOW = 10
T = 4096
BLK = 512
NW = 32
WIN_PER_W = T // NW
LEAD = 16
STAGE_ROWS = WIN_PER_W + 2 * LEAD


def _dense_body(ori_ref, w_ref, al_ref, ar_ref, bias_ref, hb_ref, el_ref, er_ref):
    h = jnp.dot(ori_ref[...], w_ref[...], preferred_element_type=jnp.float32)
    hb_ref[...] = h + bias_ref[...]
    alr = jnp.concatenate([al_ref[...], ar_ref[...]], axis=0)
    dn = (((1,), (1,)), ((), ()))
    elr = lax.dot_general(alr, h, dn, precision=lax.Precision.HIGHEST,
                          preferred_element_type=jnp.float32)
    el_ref[...] = elr[0:1].reshape(T)
    er_ref[...] = elr[1:2].reshape(T)


def _sc_agg_body(hb_hbm, el_hbm, er_hbm, out_hbm, h_v, el_v, er_v, alpha_v, out_v,
                 sem_h, sem_s):
    wid = lax.axis_index("s") * 2 + lax.axis_index("c")
    base = wid * WIN_PER_W

    @pl.when(wid != 0)
    def _():
        pltpu.sync_copy(hb_hbm.at[pl.ds(base - LEAD, LEAD)],
                        h_v.at[pl.ds(0, LEAD)])
        pltpu.sync_copy(el_hbm.at[pl.ds(base - LEAD, LEAD)],
                        el_v.at[pl.ds(0, LEAD)])

    el_copy = pltpu.async_copy(
        el_hbm.at[pl.ds(base, STAGE_ROWS - LEAD)],
        el_v.at[pl.ds(LEAD, STAGE_ROWS - LEAD)], sem_s)
    er_copy = pltpu.async_copy(
        er_hbm.at[pl.ds(base, WIN_PER_W)], er_v, sem_s)
    h_copy = pltpu.async_copy(
        hb_hbm.at[pl.ds(base, STAGE_ROWS - LEAD)],
        h_v.at[pl.ds(LEAD, STAGE_ROWS - LEAD)], sem_h)

    el_copy.wait()
    er_copy.wait()

    @pl.when(wid == 0)
    def _():
        first = el_v[pl.ds(LEAD, 16)]
        el_v[pl.ds(0, 16)] = first.at[jnp.full((16,), 0, jnp.int32)].get(
            mode="promise_in_bounds")

    for g in range(WIN_PER_W // 16):
        t0 = g * 16
        er9 = er_v[pl.ds(t0, 16)]
        scores = []
        for i in range(WINDOW):
            s = el_v[pl.ds(t0 + i + 7, 16)] + er9
            scores.append(jnp.where(s > 0, s, 0.2 * s))
        m = scores[0]
        for i in range(1, WINDOW):
            m = jnp.maximum(m, scores[i])
        ees = [jnp.exp(s - m) for s in scores]
        denom = ees[0]
        for i in range(1, WINDOW):
            denom = denom + ees[i]
        inv = 1.0 / denom
        for i in range(WINDOW):
            alpha_v[i, pl.ds(t0, 16)] = ees[i] * inv

    h_copy.wait()

    @pl.when(wid == 0)
    def _():
        for c in range(N_FEATURES // 16):
            row0 = h_v[LEAD, pl.ds(c * 16, 16)]
            for r in range(LEAD - WINDOW + 1, LEAD):
                h_v[r, pl.ds(c * 16, 16)] = row0

    def agg_group(g):
        t0 = g * 16
        av = [alpha_v[i, pl.ds(t0, 16)] for i in range(WINDOW)]
        for tb in range(4):
            ab = [[av[i].at[jnp.full((16,), tb * 4 + u, jnp.int32)].get(
                      mode="promise_in_bounds") for i in range(WINDOW)]
                  for u in range(4)]
            for c in range(N_FEATURES // 16):
                rows = [h_v[t0 + tb * 4 + r + 7, pl.ds(c * 16, 16)]
                        for r in range(4 + WINDOW - 1)]
                for u in range(4):
                    acc = ab[u][0] * rows[u]
                    for i in range(1, WINDOW):
                        acc = acc + ab[u][i] * rows[u + i]
                    out_v[t0 + tb * 4 + u, pl.ds(c * 16, 16)] = acc

    half = WIN_PER_W // 2
    plsc.parallel_loop(0, WIN_PER_W // 32, 1)(agg_group)
    out0 = pltpu.async_copy(out_v.at[pl.ds(0, half)],
                            out_hbm.at[pl.ds(base, half)], sem_s)
    plsc.parallel_loop(WIN_PER_W // 32, WIN_PER_W // 16, 1)(agg_group)
    out0.wait()
    pltpu.sync_copy(out_v.at[pl.ds(half, half)],
                    out_hbm.at[pl.ds(base + half, half)])


_sc_agg = functools.partial(
    pl.kernel,
    out_type=jax.ShapeDtypeStruct((T, N_FEATURES), jnp.float32),
    mesh=plsc.VectorSubcoreMesh(core_axis_name="c", subcore_axis_name="s"),
    scratch_types=[
        pltpu.VMEM((STAGE_ROWS, N_FEATURES), jnp.float32),
        pltpu.VMEM((STAGE_ROWS,), jnp.float32),
        pltpu.VMEM((WIN_PER_W,), jnp.float32),
        pltpu.VMEM((WINDOW, WIN_PER_W), jnp.float32),
        pltpu.VMEM((WIN_PER_W, N_FEATURES), jnp.float32),
        pltpu.SemaphoreType.DMA,
        pltpu.SemaphoreType.DMA,
    ],
)(_sc_agg_body)


def kernel(ori_feats, W, attn_l, attn_r, bias):
    hb, el, er = pl.pallas_call(
        _dense_body,
        out_shape=[
            jax.ShapeDtypeStruct((T, N_FEATURES), jnp.float32),
            jax.ShapeDtypeStruct((T,), jnp.float32),
            jax.ShapeDtypeStruct((T,), jnp.float32),
        ],
        in_specs=[pl.BlockSpec(memory_space=pltpu.VMEM)] * 5,
        out_specs=[pl.BlockSpec(memory_space=pltpu.VMEM)] * 3,
    )(ori_feats, W, attn_l.reshape(1, N_FEATURES), attn_r.reshape(1, N_FEATURES),
      bias.reshape(1, N_FEATURES))

    out = _sc_agg(hb, el, er)
    return out[:, None, :]


# speedup vs baseline: 1.1301x; 1.1301x over previous
import functools

import jax
import jax.numpy as jnp
from jax import lax
from jax.experimental import pallas as pl
from jax.experimental.pallas import tpu as pltpu
from jax.experimental.pallas import tpu_sc as plsc

N_FEATURES = 128
WINDOW = 10
T = 4096
NW = 32
WIN_PER_W = T // NW
LEAD = 16
STAGE_ROWS = WIN_PER_W + 2 * LEAD


def _dense_body(ori_ref, w_ref, al_ref, ar_ref, bias_ref, hb_ref, el_ref, er_ref):
    h = jnp.dot(ori_ref[...], w_ref[...], preferred_element_type=jnp.float32)
    hb_ref[...] = h + bias_ref[...]
    alr = jnp.concatenate([al_ref[...], ar_ref[...]], axis=0)
    dn = (((1,), (1,)), ((), ()))
    elr = lax.dot_general(alr, h, dn, precision=lax.Precision.HIGHEST,
                          preferred_element_type=jnp.float32)
    el_ref[...] = elr[0:1].reshape(T)
    er_ref[...] = elr[1:2].reshape(T)


def _sc_agg_body(hb_hbm, el_hbm, er_hbm, out_hbm, h_v, el_v, er_v, alpha_v, out_v,
                 sem_h, sem_s):
    wid = lax.axis_index("s") * 2 + lax.axis_index("c")
    base = wid * WIN_PER_W

    h_copy = pltpu.async_copy(
        hb_hbm.at[pl.ds(base, STAGE_ROWS - LEAD)],
        h_v.at[pl.ds(LEAD, STAGE_ROWS - LEAD)], sem_h)
    el_copy = pltpu.async_copy(
        el_hbm.at[pl.ds(base, STAGE_ROWS - LEAD)],
        el_v.at[pl.ds(LEAD, STAGE_ROWS - LEAD)], sem_s)
    er_copy = pltpu.async_copy(
        er_hbm.at[pl.ds(base, WIN_PER_W)], er_v, sem_s)

    @pl.when(wid != 0)
    def _():
        pltpu.sync_copy(hb_hbm.at[pl.ds(base - LEAD, LEAD)],
                        h_v.at[pl.ds(0, LEAD)])
        pltpu.sync_copy(el_hbm.at[pl.ds(base - LEAD, LEAD)],
                        el_v.at[pl.ds(0, LEAD)])

    el_copy.wait()
    er_copy.wait()

    @pl.when(wid == 0)
    def _():
        first = el_v[pl.ds(LEAD, 16)]
        el_v[pl.ds(0, 16)] = first.at[jnp.full((16,), 0, jnp.int32)].get(
            mode="promise_in_bounds")

    for g in range(WIN_PER_W // 16):
        t0 = g * 16
        er9 = er_v[pl.ds(t0, 16)]
        scores = []
        for i in range(WINDOW):
            s = el_v[pl.ds(t0 + i + 7, 16)] + er9
            scores.append(jnp.where(s > 0, s, 0.2 * s))
        m = scores[0]
        for i in range(1, WINDOW):
            m = jnp.maximum(m, scores[i])
        ees = [jnp.exp(s - m) for s in scores]
        denom = ees[0]
        for i in range(1, WINDOW):
            denom = denom + ees[i]
        inv = 1.0 / denom
        for i in range(WINDOW):
            alpha_v[i, pl.ds(t0, 16)] = ees[i] * inv

    h_copy.wait()

    @pl.when(wid == 0)
    def _():
        for c in range(N_FEATURES // 16):
            row0 = h_v[LEAD, pl.ds(c * 16, 16)]
            for r in range(LEAD - WINDOW + 1, LEAD):
                h_v[r, pl.ds(c * 16, 16)] = row0

    @plsc.parallel_loop(0, WIN_PER_W // 16, 1)
    def body(g):
        t0 = g * 16
        av = [alpha_v[i, pl.ds(t0, 16)] for i in range(WINDOW)]
        for tb in range(4):
            ab = [[av[i].at[jnp.full((16,), tb * 4 + u, jnp.int32)].get(
                      mode="promise_in_bounds") for i in range(WINDOW)]
                  for u in range(4)]
            for c in range(N_FEATURES // 16):
                rows = [h_v[t0 + tb * 4 + r + 7, pl.ds(c * 16, 16)]
                        for r in range(4 + WINDOW - 1)]
                for u in range(4):
                    acc = ab[u][0] * rows[u]
                    for i in range(1, WINDOW):
                        acc = acc + ab[u][i] * rows[u + i]
                    out_v[t0 + tb * 4 + u, pl.ds(c * 16, 16)] = acc

    pltpu.sync_copy(out_v, out_hbm.at[pl.ds(base, WIN_PER_W)])


_sc_agg = functools.partial(
    pl.kernel,
    out_type=jax.ShapeDtypeStruct((T, N_FEATURES), jnp.float32),
    mesh=plsc.VectorSubcoreMesh(core_axis_name="c", subcore_axis_name="s"),
    scratch_types=[
        pltpu.VMEM((STAGE_ROWS, N_FEATURES), jnp.float32),
        pltpu.VMEM((STAGE_ROWS,), jnp.float32),
        pltpu.VMEM((WIN_PER_W,), jnp.float32),
        pltpu.VMEM((WINDOW, WIN_PER_W), jnp.float32),
        pltpu.VMEM((WIN_PER_W, N_FEATURES), jnp.float32),
        pltpu.SemaphoreType.DMA,
        pltpu.SemaphoreType.DMA,
    ],
)(_sc_agg_body)


def kernel(ori_feats, W, attn_l, attn_r, bias):
    hb, el, er = pl.pallas_call(
        _dense_body,
        out_shape=[
            jax.ShapeDtypeStruct((T, N_FEATURES), jnp.float32),
            jax.ShapeDtypeStruct((T,), jnp.float32),
            jax.ShapeDtypeStruct((T,), jnp.float32),
        ],
        in_specs=[pl.BlockSpec(memory_space=pltpu.VMEM)] * 5,
        out_specs=[pl.BlockSpec(memory_space=pltpu.VMEM)] * 3,
    )(ori_feats, W, attn_l.reshape(1, N_FEATURES), attn_r.reshape(1, N_FEATURES),
      bias.reshape(1, N_FEATURES))

    out = _sc_agg(hb, el, er)
    return out[:, None, :]
